# Initial kernel scaffold; baseline (speedup 1.0000x reference)
import jax
import jax.numpy as jnp
from jax.experimental import pallas as pl


def kernel(h, Wn, bn, We, be, Wc_gcn, bc_gcn, gc_gamma, gc_beta, Wo_gcn, bo_gcn, go_gamma, go_beta, Wctx, bctx, Wobj, bobj, Wco, bco, edge_index, graph_ids):
    raise NotImplementedError("write your pallas kernel here")


# speedup vs baseline: 4.4796x; 4.4796x over previous
import functools

import jax
import jax.numpy as jnp
from jax import lax
from jax.experimental import pallas as pl
from jax.experimental.pallas import tpu as pltpu
from jax.experimental.pallas import tpu_sc as plsc

_NUM_GRAPHS = 128
_LANES = 16
_TILES = 16
_CHUNK = 128


def _deg_body(num_chunks, npt, idx2_hbm, z1_hbm, deg_hbm, hist, idx, ones):
  cid = lax.axis_index("c")
  sid = lax.axis_index("s")
  base = sid * npt

  pltpu.sync_copy(z1_hbm, hist.at[pl.ds(base, npt)])
  for j in range(_CHUNK // _LANES):
    ones[pl.ds(j * _LANES, _LANES)] = jnp.full((_LANES,), 1.0, jnp.float32)
  plsc.subcore_barrier()

  def body(k, carry):
    pltpu.sync_copy(idx2_hbm.at[cid, sid, k], idx)
    pltpu.sync_copy(ones, hist.at[idx], add=True)
    return carry

  lax.fori_loop(0, num_chunks, body, 0)
  plsc.subcore_barrier()
  pltpu.sync_copy(hist.at[pl.ds(base, npt)], deg_hbm.at[cid, pl.ds(base, npt)])


def _prep_body(h_ref, Wn_ref, bn_ref, We_ref, be_ref, deg_ref,
               psc_ref, pso_ref, da_ref, db_ref):
  hh = h_ref[...]
  d = hh.shape[1]
  att = jax.nn.softmax(
      jnp.dot(hh, Wn_ref[...], preferred_element_type=jnp.float32)
      + bn_ref[...][None, :], axis=-1)
  u = jnp.dot(hh, We_ref[...][:d], preferred_element_type=jnp.float32)
  v = jnp.dot(hh, We_ref[...][d:], preferred_element_type=jnp.float32)
  onorm = lax.rsqrt(jnp.maximum(deg_ref[0], 1.0))
  psc_ref[...] = att[:, 0] * onorm
  pso_ref[...] = att[:, 1] * onorm
  da_ref[...] = u[:, 1] - u[:, 0]
  db_ref[...] = v[:, 1] - v[:, 0] + (be_ref[1] - be_ref[0])


def _main_body(num_chunks, npt, d,
               hp_hbm, idx2_hbm, psc_hbm, pso_hbm, da_hbm, db_hbm, z2_hbm,
               agg_hbm,
               acc, idxr, idxc, rows, pv1, pv2, dav, dbv, wbuf, sem):
  cid = lax.axis_index("c")
  sid = lax.axis_index("s")
  base = sid * npt

  pltpu.sync_copy(z2_hbm, acc.at[pl.ds(base, npt)])
  plsc.subcore_barrier()
  isc0 = jnp.broadcast_to(cid == 0, (_LANES,))

  def body(k, carry):
    pltpu.sync_copy(idx2_hbm.at[0, sid, k], idxr)
    pltpu.sync_copy(idx2_hbm.at[1, sid, k], idxc)
    c1 = pltpu.async_copy(hp_hbm.at[idxr], rows, sem)
    c2 = pltpu.async_copy(psc_hbm.at[idxr], pv1, sem)
    c3 = pltpu.async_copy(pso_hbm.at[idxr], pv2, sem)
    c4 = pltpu.async_copy(da_hbm.at[idxr], dav, sem)
    c5 = pltpu.async_copy(db_hbm.at[idxc], dbv, sem)
    c1.wait()
    c2.wait()
    c3.wait()
    c4.wait()
    c5.wait()
    for g in range(_CHUNK // _LANES):
      sl = pl.ds(g * _LANES, _LANES)
      z = jnp.exp(dav[sl] + dbv[sl])
      inv = 1.0 / (1.0 + z)
      wbuf[sl] = jnp.where(isc0, pv1[sl] * inv, pv2[sl] * (1.0 - inv))

    def row_body(r, rcarry):
      sb = plsc.load_gather(wbuf, [jnp.broadcast_to(r, (_LANES,))])
      for j in range(d // _LANES):
        slj = pl.ds(j * _LANES, _LANES)
        rows[r, slj] = rows[r, slj] * sb
      return rcarry

    lax.fori_loop(0, _CHUNK, row_body, 0)
    pltpu.sync_copy(rows, acc.at[idxc], add=True)
    return carry

  lax.fori_loop(0, num_chunks, body, 0)
  plsc.subcore_barrier()
  pltpu.sync_copy(acc.at[pl.ds(base, npt)], agg_hbm.at[cid, pl.ds(base, npt)])


def _post_body(aggc_ref, aggo_ref, degin_ref, gids_ref,
               Wc_ref, bc_ref, gcg_ref, gcb_ref,
               Wo_ref, bo_ref, gog_ref, gob_ref,
               Wctx_ref, bctx_ref, Wobj_ref, bobj_ref, Wco_ref, bco_ref,
               c_ref, o_ref, co_ref):
  n = aggc_ref.shape[0]
  inorm = lax.rsqrt(jnp.maximum(degin_ref[...], 1.0))

  def gcn_tail(agg, W, b, gamma, beta):
    rst = jnp.dot(agg * inorm[:, None], W,
                  preferred_element_type=jnp.float32) + b[None, :]
    mu = jnp.mean(rst, axis=0, keepdims=True)
    var = jnp.mean((rst - mu) ** 2, axis=0, keepdims=True)
    hbn = (rst - mu) * lax.rsqrt(var + 1e-5) * gamma[None, :] + beta[None, :]
    return jnp.maximum(hbn, 0.0)

  xc = gcn_tail(aggc_ref[...], Wc_ref[...], bc_ref[...], gcg_ref[...],
                gcb_ref[...])
  xo = gcn_tail(aggo_ref[...], Wo_ref[...], bo_ref[...], gog_ref[...],
                gob_ref[...])

  gids = gids_ref[...]
  onehot = (gids[None, :] ==
            lax.broadcasted_iota(jnp.int32, (_NUM_GRAPHS, n), 0)
            ).astype(jnp.float32)
  cnt = jnp.maximum(jnp.sum(onehot, axis=1), 1.0)
  hgc = jnp.dot(onehot, xc, preferred_element_type=jnp.float32) / cnt[:, None]
  hgo = jnp.dot(onehot, xo, preferred_element_type=jnp.float32) / cnt[:, None]
  avg_ctx = jnp.mean(hgc, axis=0, keepdims=True)
  c_ref[...] = jnp.dot(hgc, Wctx_ref[...],
                       preferred_element_type=jnp.float32) + bctx_ref[...][None, :]
  o_ref[...] = jnp.dot(hgo, Wobj_ref[...],
                       preferred_element_type=jnp.float32) + bobj_ref[...][None, :]
  co_ref[...] = jnp.dot(avg_ctx + hgo, Wco_ref[...],
                        preferred_element_type=jnp.float32) + bco_ref[...][None, :]


def kernel(h, Wn, bn, We, be, Wc_gcn, bc_gcn, gc_gamma, gc_beta,
           Wo_gcn, bo_gcn, go_gamma, go_beta, Wctx, bctx, Wobj, bobj,
           Wco, bco, edge_index, graph_ids):
  n, d = h.shape
  e = edge_index.shape[1]
  f32 = jnp.float32

  np_pad = ((n + _TILES * _LANES - 1) // (_TILES * _LANES)) * (_TILES * _LANES)
  num_chunks = (e + _TILES * _CHUNK - 1) // (_TILES * _CHUNK)
  pe = _TILES * num_chunks * _CHUNK
  npt = np_pad // _TILES

  hp = jnp.pad(h, ((0, np_pad - n), (0, 0)))
  pad_idx = n + (jnp.arange(pe - e, dtype=jnp.int32) % (np_pad - n))
  rowr = jnp.concatenate([edge_index[0], pad_idx]).reshape(
      _TILES, num_chunks, _CHUNK)
  colr = jnp.concatenate([edge_index[1], pad_idx]).reshape(
      _TILES, num_chunks, _CHUNK)
  idx2 = jnp.stack([rowr, colr])

  z1 = jnp.zeros((npt,), f32)
  z2 = jnp.zeros((npt, d), f32)

  mesh = plsc.VectorSubcoreMesh(core_axis_name="c", subcore_axis_name="s")

  deg = pl.kernel(
      functools.partial(_deg_body, num_chunks, npt),
      out_type=jax.ShapeDtypeStruct((2, np_pad), f32),
      mesh=mesh,
      scratch_types=[
          pltpu.VMEM_SHARED((np_pad,), f32),
          pltpu.VMEM((_CHUNK,), jnp.int32),
          pltpu.VMEM((_CHUNK,), f32),
      ],
  )(idx2, z1)

  tab_shape = jax.ShapeDtypeStruct((np_pad,), f32)
  psc, pso, da, db = pl.pallas_call(
      _prep_body,
      out_shape=(tab_shape, tab_shape, tab_shape, tab_shape),
  )(hp, Wn, bn, We, be, deg)

  agg = pl.kernel(
      functools.partial(_main_body, num_chunks, npt, d),
      out_type=jax.ShapeDtypeStruct((2, np_pad, d), f32),
      mesh=mesh,
      compiler_params=pltpu.CompilerParams(needs_layout_passes=False),
      scratch_types=[
          pltpu.VMEM_SHARED((np_pad, d), f32),
          pltpu.VMEM((_CHUNK,), jnp.int32),
          pltpu.VMEM((_CHUNK,), jnp.int32),
          pltpu.VMEM((_CHUNK, d), f32),
          pltpu.VMEM((_CHUNK,), f32),
          pltpu.VMEM((_CHUNK,), f32),
          pltpu.VMEM((_CHUNK,), f32),
          pltpu.VMEM((_CHUNK,), f32),
          pltpu.VMEM((_CHUNK,), f32),
          pltpu.SemaphoreType.DMA,
      ],
  )(hp, idx2, psc, pso, da, db, z2)

  out_shape = jax.ShapeDtypeStruct((_NUM_GRAPHS, Wctx.shape[1]), f32)
  c_out, o_out, co_out = pl.pallas_call(
      _post_body,
      out_shape=(out_shape, out_shape, out_shape),
  )(agg[0, :n], agg[1, :n], deg[1, :n], graph_ids,
    Wc_gcn, bc_gcn, gc_gamma, gc_beta,
    Wo_gcn, bo_gcn, go_gamma, go_beta,
    Wctx, bctx, Wobj, bobj, Wco, bco)
  return (c_out, o_out, co_out)


# speedup vs baseline: 6.1157x; 1.3652x over previous
import functools

import jax
import jax.numpy as jnp
from jax import lax
from jax.experimental import pallas as pl
from jax.experimental.pallas import tpu as pltpu
from jax.experimental.pallas import tpu_sc as plsc

_NUM_GRAPHS = 128
_LANES = 16
_TILES = 16
_CHUNK = 128
_GROUP = 2


def _deg_body(ngroups, npt, idx2_hbm, z1_hbm, deg_hbm, hist, idx, ones, sem):
  cid = lax.axis_index("c")
  sid = lax.axis_index("s")
  base = sid * npt

  pltpu.sync_copy(z1_hbm, hist.at[pl.ds(base, npt)])
  for j in range(_CHUNK // _LANES):
    ones[pl.ds(j * _LANES, _LANES)] = jnp.full((_LANES,), 1.0, jnp.float32)
  plsc.subcore_barrier()

  def body(g, carry):
    pltpu.sync_copy(idx2_hbm.at[cid, sid, pl.ds(g * _GROUP, _GROUP)], idx)
    adds = [
        pltpu.async_copy(ones, hist.at[idx.at[q]], sem, add=True)
        for q in range(_GROUP)
    ]
    for a in adds:
      a.wait()
    return carry

  lax.fori_loop(0, ngroups, body, 0)
  plsc.subcore_barrier()
  pltpu.sync_copy(hist.at[pl.ds(base, npt)], deg_hbm.at[cid, pl.ds(base, npt)])


def _prep_body(h_ref, Wn_ref, bn_ref, We_ref, be_ref, deg_ref,
               psc_ref, pso_ref, da_ref, db_ref):
  hh = h_ref[...]
  d = hh.shape[1]
  att = jax.nn.softmax(
      jnp.dot(hh, Wn_ref[...], preferred_element_type=jnp.float32)
      + bn_ref[...][None, :], axis=-1)
  u = jnp.dot(hh, We_ref[...][:d], preferred_element_type=jnp.float32)
  v = jnp.dot(hh, We_ref[...][d:], preferred_element_type=jnp.float32)
  onorm = lax.rsqrt(jnp.maximum(deg_ref[0], 1.0))
  psc_ref[...] = att[:, 0] * onorm
  pso_ref[...] = att[:, 1] * onorm
  da_ref[...] = u[:, 1] - u[:, 0]
  db_ref[...] = v[:, 1] - v[:, 0] + (be_ref[1] - be_ref[0])


def _main_body(ngroups, npt, d,
               hp_hbm, idx2_hbm, psc_hbm, pso_hbm, da_hbm, db_hbm, z2_hbm,
               agg_hbm,
               acc, idxr, idxc, rows, pv1, pv2, dav, dbv, wbuf,
               sem_g, sem_s):
  cid = lax.axis_index("c")
  sid = lax.axis_index("s")
  base = sid * npt
  gc = _GROUP * _CHUNK

  pltpu.sync_copy(z2_hbm, acc.at[pl.ds(base, npt)])
  plsc.subcore_barrier()
  isc0 = jnp.broadcast_to(cid == 0, (_LANES,))

  def body(g, carry):
    pltpu.sync_copy(idx2_hbm.at[0, sid, pl.ds(g * _GROUP, _GROUP)], idxr)
    pltpu.sync_copy(idx2_hbm.at[1, sid, pl.ds(g * _GROUP, _GROUP)], idxc)
    gathers = []
    for q in range(_GROUP):
      rsl = pl.ds(q * _CHUNK, _CHUNK)
      gathers.append(
          (pltpu.async_copy(hp_hbm.at[idxr.at[q]], rows.at[rsl], sem_g),
           pltpu.async_copy(psc_hbm.at[idxr.at[q]], pv1.at[rsl], sem_g),
           pltpu.async_copy(pso_hbm.at[idxr.at[q]], pv2.at[rsl], sem_g),
           pltpu.async_copy(da_hbm.at[idxr.at[q]], dav.at[rsl], sem_g),
           pltpu.async_copy(db_hbm.at[idxc.at[q]], dbv.at[rsl], sem_g)))
    scatters = []
    for q in range(_GROUP):
      for c in gathers[q]:
        c.wait()
      for j in range(_CHUNK // _LANES):
        sl = pl.ds(q * _CHUNK + j * _LANES, _LANES)
        z = jnp.exp(dav[sl] + dbv[sl])
        inv = 1.0 / (1.0 + z)
        wbuf[sl] = jnp.where(isc0, pv1[sl] * inv, pv2[sl] * (1.0 - inv))

      def row_body(r, rcarry):
        sb = plsc.load_gather(wbuf, [jnp.broadcast_to(r, (_LANES,))])
        for j in range(d // _LANES):
          slj = pl.ds(j * _LANES, _LANES)
          rows[r, slj] = rows[r, slj] * sb
        return rcarry

      lax.fori_loop(q * _CHUNK, (q + 1) * _CHUNK, row_body, 0)
      scatters.append(
          pltpu.async_copy(rows.at[pl.ds(q * _CHUNK, _CHUNK)],
                           acc.at[idxc.at[q]], sem_s, add=True))
    for s in scatters:
      s.wait()
    return carry

  lax.fori_loop(0, ngroups, body, 0)
  plsc.subcore_barrier()
  pltpu.sync_copy(acc.at[pl.ds(base, npt)], agg_hbm.at[cid, pl.ds(base, npt)])


def _post_body(aggc_ref, aggo_ref, degin_ref, gids_ref,
               Wc_ref, bc_ref, gcg_ref, gcb_ref,
               Wo_ref, bo_ref, gog_ref, gob_ref,
               Wctx_ref, bctx_ref, Wobj_ref, bobj_ref, Wco_ref, bco_ref,
               c_ref, o_ref, co_ref):
  n = aggc_ref.shape[0]
  inorm = lax.rsqrt(jnp.maximum(degin_ref[...], 1.0))

  def gcn_tail(agg, W, b, gamma, beta):
    rst = jnp.dot(agg * inorm[:, None], W,
                  preferred_element_type=jnp.float32) + b[None, :]
    mu = jnp.mean(rst, axis=0, keepdims=True)
    var = jnp.mean((rst - mu) ** 2, axis=0, keepdims=True)
    hbn = (rst - mu) * lax.rsqrt(var + 1e-5) * gamma[None, :] + beta[None, :]
    return jnp.maximum(hbn, 0.0)

  xc = gcn_tail(aggc_ref[...], Wc_ref[...], bc_ref[...], gcg_ref[...],
                gcb_ref[...])
  xo = gcn_tail(aggo_ref[...], Wo_ref[...], bo_ref[...], gog_ref[...],
                gob_ref[...])

  gids = gids_ref[...]
  onehot = (gids[None, :] ==
            lax.broadcasted_iota(jnp.int32, (_NUM_GRAPHS, n), 0)
            ).astype(jnp.float32)
  cnt = jnp.maximum(jnp.sum(onehot, axis=1), 1.0)
  hgc = jnp.dot(onehot, xc, preferred_element_type=jnp.float32) / cnt[:, None]
  hgo = jnp.dot(onehot, xo, preferred_element_type=jnp.float32) / cnt[:, None]
  avg_ctx = jnp.mean(hgc, axis=0, keepdims=True)
  c_ref[...] = jnp.dot(hgc, Wctx_ref[...],
                       preferred_element_type=jnp.float32) + bctx_ref[...][None, :]
  o_ref[...] = jnp.dot(hgo, Wobj_ref[...],
                       preferred_element_type=jnp.float32) + bobj_ref[...][None, :]
  co_ref[...] = jnp.dot(avg_ctx + hgo, Wco_ref[...],
                        preferred_element_type=jnp.float32) + bco_ref[...][None, :]


def kernel(h, Wn, bn, We, be, Wc_gcn, bc_gcn, gc_gamma, gc_beta,
           Wo_gcn, bo_gcn, go_gamma, go_beta, Wctx, bctx, Wobj, bobj,
           Wco, bco, edge_index, graph_ids):
  n, d = h.shape
  e = edge_index.shape[1]
  f32 = jnp.float32

  np_pad = ((n + _TILES * _LANES - 1) // (_TILES * _LANES)) * (_TILES * _LANES)
  gsz = _TILES * _CHUNK * _GROUP
  ngroups = (e + gsz - 1) // gsz
  num_chunks = ngroups * _GROUP
  pe = _TILES * num_chunks * _CHUNK
  npt = np_pad // _TILES

  hp = jnp.pad(h, ((0, np_pad - n), (0, 0)))
  pad_idx = n + (jnp.arange(pe - e, dtype=jnp.int32) % (np_pad - n))
  rowr = jnp.concatenate([edge_index[0], pad_idx]).reshape(
      _TILES, num_chunks, _CHUNK)
  colr = jnp.concatenate([edge_index[1], pad_idx]).reshape(
      _TILES, num_chunks, _CHUNK)
  idx2 = jnp.stack([rowr, colr])

  z1 = jnp.zeros((npt,), f32)
  z2 = jnp.zeros((npt, d), f32)

  mesh = plsc.VectorSubcoreMesh(core_axis_name="c", subcore_axis_name="s")

  deg = pl.kernel(
      functools.partial(_deg_body, ngroups, npt),
      out_type=jax.ShapeDtypeStruct((2, np_pad), f32),
      mesh=mesh,
      scratch_types=[
          pltpu.VMEM_SHARED((np_pad,), f32),
          pltpu.VMEM((_GROUP, _CHUNK), jnp.int32),
          pltpu.VMEM((_CHUNK,), f32),
          pltpu.SemaphoreType.DMA,
      ],
  )(idx2, z1)

  tab_shape = jax.ShapeDtypeStruct((np_pad,), f32)
  psc, pso, da, db = pl.pallas_call(
      _prep_body,
      out_shape=(tab_shape, tab_shape, tab_shape, tab_shape),
  )(hp, Wn, bn, We, be, deg)

  gc = _GROUP * _CHUNK
  agg = pl.kernel(
      functools.partial(_main_body, ngroups, npt, d),
      out_type=jax.ShapeDtypeStruct((2, np_pad, d), f32),
      mesh=mesh,
      compiler_params=pltpu.CompilerParams(needs_layout_passes=False),
      scratch_types=[
          pltpu.VMEM_SHARED((np_pad, d), f32),
          pltpu.VMEM((_GROUP, _CHUNK), jnp.int32),
          pltpu.VMEM((_GROUP, _CHUNK), jnp.int32),
          pltpu.VMEM((gc, d), f32),
          pltpu.VMEM((gc,), f32),
          pltpu.VMEM((gc,), f32),
          pltpu.VMEM((gc,), f32),
          pltpu.VMEM((gc,), f32),
          pltpu.VMEM((gc,), f32),
          pltpu.SemaphoreType.DMA,
          pltpu.SemaphoreType.DMA,
      ],
  )(hp, idx2, psc, pso, da, db, z2)

  out_shape = jax.ShapeDtypeStruct((_NUM_GRAPHS, Wctx.shape[1]), f32)
  c_out, o_out, co_out = pl.pallas_call(
      _post_body,
      out_shape=(out_shape, out_shape, out_shape),
  )(agg[0, :n], agg[1, :n], deg[1, :n], graph_ids,
    Wc_gcn, bc_gcn, gc_gamma, gc_beta,
    Wo_gcn, bo_gcn, go_gamma, go_beta,
    Wctx, bctx, Wobj, bobj, Wco, bco)
  return (c_out, o_out, co_out)


# speedup vs baseline: 7.5109x; 1.2281x over previous
import functools

import jax
import jax.numpy as jnp
from jax import lax
from jax.experimental import pallas as pl
from jax.experimental.pallas import tpu as pltpu
from jax.experimental.pallas import tpu_sc as plsc

_NUM_GRAPHS = 128
_LANES = 16
_TILES = 16
_CHUNK = 128
_GROUP = 2


def _deg_body(ngroups, npt, idx2_hbm, z1_hbm, deg_hbm, hist, idx, ones, sem):
  cid = lax.axis_index("c")
  sid = lax.axis_index("s")
  base = sid * npt

  pltpu.sync_copy(z1_hbm, hist.at[pl.ds(base, npt)])
  for j in range(_CHUNK // _LANES):
    ones[pl.ds(j * _LANES, _LANES)] = jnp.full((_LANES,), 1.0, jnp.float32)
  plsc.subcore_barrier()

  def body(g, carry):
    pltpu.sync_copy(idx2_hbm.at[cid, sid, pl.ds(g * _GROUP, _GROUP)], idx)
    adds = [
        pltpu.async_copy(ones, hist.at[idx.at[q]], sem, add=True)
        for q in range(_GROUP)
    ]
    for a in adds:
      a.wait()
    return carry

  lax.fori_loop(0, ngroups, body, 0)
  plsc.subcore_barrier()
  pltpu.sync_copy(hist.at[pl.ds(base, npt)], deg_hbm.at[cid, pl.ds(base, npt)])


def _prep_body(h_ref, Wn_ref, bn_ref, We_ref, be_ref, deg_ref,
               psc_ref, pso_ref, da_ref, db_ref):
  hh = h_ref[...]
  d = hh.shape[1]
  att = jax.nn.softmax(
      jnp.dot(hh, Wn_ref[...], preferred_element_type=jnp.float32)
      + bn_ref[...][None, :], axis=-1)
  u = jnp.dot(hh, We_ref[...][:d], preferred_element_type=jnp.float32)
  v = jnp.dot(hh, We_ref[...][d:], preferred_element_type=jnp.float32)
  onorm = lax.rsqrt(jnp.maximum(deg_ref[0], 1.0))
  psc_ref[...] = att[:, 0] * onorm
  pso_ref[...] = att[:, 1] * onorm
  da_ref[...] = u[:, 1] - u[:, 0]
  db_ref[...] = v[:, 1] - v[:, 0] + (be_ref[1] - be_ref[0])


def _main_body(num_chunks, npt, d,
               hp_hbm, idxp_hbm, psc_hbm, pso_hbm, da_hbm, db_hbm, z2_hbm,
               agg_hbm,
               acc, ii0, ii1, rows0, rows1,
               pva0, pvb0, da0, db0, pva1, pvb1, da1, db1, wbuf,
               sem_g, sem_s0, sem_s1):
  cid = lax.axis_index("c")
  sid = lax.axis_index("s")
  base = sid * npt
  npairs = num_chunks // 2

  pltpu.sync_copy(z2_hbm, acc.at[pl.ds(base, npt)])
  plsc.subcore_barrier()
  isc0 = jnp.broadcast_to(cid == 0, (_LANES,))

  def fire_gathers(ii, rows, pva, pvb, dav, dbv):
    pltpu.async_copy(hp_hbm.at[ii.at[0]], rows, sem_g)
    pltpu.async_copy(psc_hbm.at[ii.at[0]], pva, sem_g)
    pltpu.async_copy(pso_hbm.at[ii.at[0]], pvb, sem_g)
    pltpu.async_copy(da_hbm.at[ii.at[0]], dav, sem_g)
    pltpu.async_copy(db_hbm.at[ii.at[1]], dbv, sem_g)

  def wait_gathers(ii, rows, pva, pvb, dav, dbv):
    pltpu.make_async_copy(hp_hbm.at[ii.at[0]], rows, sem_g).wait()
    pltpu.make_async_copy(psc_hbm.at[ii.at[0]], pva, sem_g).wait()
    pltpu.make_async_copy(pso_hbm.at[ii.at[0]], pvb, sem_g).wait()
    pltpu.make_async_copy(da_hbm.at[ii.at[0]], dav, sem_g).wait()
    pltpu.make_async_copy(db_hbm.at[ii.at[1]], dbv, sem_g).wait()

  def wait_scatter(ii, rows, sem):
    pltpu.make_async_copy(rows, acc.at[ii.at[1]], sem).wait()

  def weigh_and_scale(rows, pva, pvb, dav, dbv):
    for g in range(_CHUNK // _LANES):
      sl = pl.ds(g * _LANES, _LANES)
      z = jnp.exp(dav[sl] + dbv[sl])
      inv = 1.0 / (1.0 + z)
      wbuf[sl] = jnp.where(isc0, pva[sl] * inv, pvb[sl] * (1.0 - inv))

    def row_body(r, rcarry):
      sb = plsc.load_gather(wbuf, [jnp.broadcast_to(r, (_LANES,))])
      for j in range(d // _LANES):
        slj = pl.ds(j * _LANES, _LANES)
        rows[r, slj] = rows[r, slj] * sb
      return rcarry

    lax.fori_loop(0, _CHUNK, row_body, 0)

  pltpu.sync_copy(idxp_hbm.at[sid, 0], ii0)
  fire_gathers(ii0, rows0, pva0, pvb0, da0, db0)

  def pair(i, carry):
    wait_gathers(ii0, rows0, pva0, pvb0, da0, db0)

    @pl.when(i > 0)
    def _():
      wait_scatter(ii1, rows1, sem_s1)

    pltpu.sync_copy(idxp_hbm.at[sid, 2 * i + 1], ii1)
    fire_gathers(ii1, rows1, pva1, pvb1, da1, db1)
    weigh_and_scale(rows0, pva0, pvb0, da0, db0)
    pltpu.async_copy(rows0, acc.at[ii0.at[1]], sem_s0, add=True)

    wait_gathers(ii1, rows1, pva1, pvb1, da1, db1)
    wait_scatter(ii0, rows0, sem_s0)

    @pl.when(i < npairs - 1)
    def _():
      pltpu.sync_copy(idxp_hbm.at[sid, 2 * i + 2], ii0)
      fire_gathers(ii0, rows0, pva0, pvb0, da0, db0)

    weigh_and_scale(rows1, pva1, pvb1, da1, db1)
    pltpu.async_copy(rows1, acc.at[ii1.at[1]], sem_s1, add=True)
    return carry

  lax.fori_loop(0, npairs, pair, 0)
  wait_scatter(ii1, rows1, sem_s1)
  plsc.subcore_barrier()
  pltpu.sync_copy(acc.at[pl.ds(base, npt)], agg_hbm.at[cid, pl.ds(base, npt)])


def _post_body(aggc_ref, aggo_ref, degin_ref, gids_ref,
               Wc_ref, bc_ref, gcg_ref, gcb_ref,
               Wo_ref, bo_ref, gog_ref, gob_ref,
               Wctx_ref, bctx_ref, Wobj_ref, bobj_ref, Wco_ref, bco_ref,
               c_ref, o_ref, co_ref):
  n = aggc_ref.shape[0]
  inorm = lax.rsqrt(jnp.maximum(degin_ref[...], 1.0))

  def gcn_tail(agg, W, b, gamma, beta):
    rst = jnp.dot(agg * inorm[:, None], W,
                  preferred_element_type=jnp.float32) + b[None, :]
    mu = jnp.mean(rst, axis=0, keepdims=True)
    var = jnp.mean((rst - mu) ** 2, axis=0, keepdims=True)
    hbn = (rst - mu) * lax.rsqrt(var + 1e-5) * gamma[None, :] + beta[None, :]
    return jnp.maximum(hbn, 0.0)

  xc = gcn_tail(aggc_ref[...], Wc_ref[...], bc_ref[...], gcg_ref[...],
                gcb_ref[...])
  xo = gcn_tail(aggo_ref[...], Wo_ref[...], bo_ref[...], gog_ref[...],
                gob_ref[...])

  gids = gids_ref[...]
  onehot = (gids[None, :] ==
            lax.broadcasted_iota(jnp.int32, (_NUM_GRAPHS, n), 0)
            ).astype(jnp.float32)
  cnt = jnp.maximum(jnp.sum(onehot, axis=1), 1.0)
  hgc = jnp.dot(onehot, xc, preferred_element_type=jnp.float32) / cnt[:, None]
  hgo = jnp.dot(onehot, xo, preferred_element_type=jnp.float32) / cnt[:, None]
  avg_ctx = jnp.mean(hgc, axis=0, keepdims=True)
  c_ref[...] = jnp.dot(hgc, Wctx_ref[...],
                       preferred_element_type=jnp.float32) + bctx_ref[...][None, :]
  o_ref[...] = jnp.dot(hgo, Wobj_ref[...],
                       preferred_element_type=jnp.float32) + bobj_ref[...][None, :]
  co_ref[...] = jnp.dot(avg_ctx + hgo, Wco_ref[...],
                        preferred_element_type=jnp.float32) + bco_ref[...][None, :]


def kernel(h, Wn, bn, We, be, Wc_gcn, bc_gcn, gc_gamma, gc_beta,
           Wo_gcn, bo_gcn, go_gamma, go_beta, Wctx, bctx, Wobj, bobj,
           Wco, bco, edge_index, graph_ids):
  n, d = h.shape
  e = edge_index.shape[1]
  f32 = jnp.float32

  np_pad = ((n + _TILES * _LANES - 1) // (_TILES * _LANES)) * (_TILES * _LANES)
  gsz = _TILES * _CHUNK * _GROUP
  ngroups = (e + gsz - 1) // gsz
  num_chunks = ngroups * _GROUP
  pe = _TILES * num_chunks * _CHUNK
  npt = np_pad // _TILES

  hp = jnp.pad(h, ((0, np_pad - n), (0, 0)))
  pad_idx = n + (jnp.arange(pe - e, dtype=jnp.int32) % (np_pad - n))
  rowr = jnp.concatenate([edge_index[0], pad_idx]).reshape(
      _TILES, num_chunks, _CHUNK)
  colr = jnp.concatenate([edge_index[1], pad_idx]).reshape(
      _TILES, num_chunks, _CHUNK)
  idx2 = jnp.stack([rowr, colr])
  idxm = jnp.stack([rowr, colr], axis=2)

  z1 = jnp.zeros((npt,), f32)
  z2 = jnp.zeros((npt, d), f32)

  mesh = plsc.VectorSubcoreMesh(core_axis_name="c", subcore_axis_name="s")

  deg = pl.kernel(
      functools.partial(_deg_body, ngroups, npt),
      out_type=jax.ShapeDtypeStruct((2, np_pad), f32),
      mesh=mesh,
      scratch_types=[
          pltpu.VMEM_SHARED((np_pad,), f32),
          pltpu.VMEM((_GROUP, _CHUNK), jnp.int32),
          pltpu.VMEM((_CHUNK,), f32),
          pltpu.SemaphoreType.DMA,
      ],
  )(idx2, z1)

  tab_shape = jax.ShapeDtypeStruct((np_pad,), f32)
  psc, pso, da, db = pl.pallas_call(
      _prep_body,
      out_shape=(tab_shape, tab_shape, tab_shape, tab_shape),
  )(hp, Wn, bn, We, be, deg)

  agg = pl.kernel(
      functools.partial(_main_body, num_chunks, npt, d),
      out_type=jax.ShapeDtypeStruct((2, np_pad, d), f32),
      mesh=mesh,
      compiler_params=pltpu.CompilerParams(needs_layout_passes=False),
      scratch_types=[
          pltpu.VMEM_SHARED((np_pad, d), f32),
          pltpu.VMEM((2, _CHUNK), jnp.int32),
          pltpu.VMEM((2, _CHUNK), jnp.int32),
          pltpu.VMEM((_CHUNK, d), f32),
          pltpu.VMEM((_CHUNK, d), f32),
          pltpu.VMEM((_CHUNK,), f32),
          pltpu.VMEM((_CHUNK,), f32),
          pltpu.VMEM((_CHUNK,), f32),
          pltpu.VMEM((_CHUNK,), f32),
          pltpu.VMEM((_CHUNK,), f32),
          pltpu.VMEM((_CHUNK,), f32),
          pltpu.VMEM((_CHUNK,), f32),
          pltpu.VMEM((_CHUNK,), f32),
          pltpu.VMEM((_CHUNK,), f32),
          pltpu.SemaphoreType.DMA,
          pltpu.SemaphoreType.DMA,
          pltpu.SemaphoreType.DMA,
      ],
  )(hp, idxm, psc, pso, da, db, z2)

  out_shape = jax.ShapeDtypeStruct((_NUM_GRAPHS, Wctx.shape[1]), f32)
  c_out, o_out, co_out = pl.pallas_call(
      _post_body,
      out_shape=(out_shape, out_shape, out_shape),
  )(agg[0, :n], agg[1, :n], deg[1, :n], graph_ids,
    Wc_gcn, bc_gcn, gc_gamma, gc_beta,
    Wo_gcn, bo_gcn, go_gamma, go_beta,
    Wctx, bctx, Wobj, bobj, Wco, bco)
  return (c_out, o_out, co_out)


# speedup vs baseline: 8.1115x; 1.0800x over previous
import functools

import jax
import jax.numpy as jnp
from jax import lax
from jax.experimental import pallas as pl
from jax.experimental.pallas import tpu as pltpu
from jax.experimental.pallas import tpu_sc as plsc

_NUM_GRAPHS = 128
_LANES = 16
_TILES = 16
_CHUNK = 128
_GROUP = 2


_DEG_WINDOW = 4


def _deg_body(num_chunks, npt, idx2_hbm, z1_hbm, deg_hbm, hist, idx, ones, sem):
  cid = lax.axis_index("c")
  sid = lax.axis_index("s")
  base = sid * npt

  pltpu.sync_copy(z1_hbm, hist.at[pl.ds(base, npt)])
  for j in range(_CHUNK // _LANES):
    ones[pl.ds(j * _LANES, _LANES)] = jnp.full((_LANES,), 1.0, jnp.float32)
  pltpu.sync_copy(idx2_hbm.at[cid, sid], idx)
  plsc.subcore_barrier()

  for k in range(_DEG_WINDOW):
    pltpu.async_copy(ones, hist.at[idx.at[k]], sem, add=True)

  def body(k, carry):
    pltpu.make_async_copy(ones, hist.at[idx.at[k]], sem).wait()

    @pl.when(k + _DEG_WINDOW < num_chunks)
    def _():
      pltpu.async_copy(ones, hist.at[idx.at[k + _DEG_WINDOW]], sem, add=True)

    return carry

  lax.fori_loop(0, num_chunks, body, 0)
  plsc.subcore_barrier()
  pltpu.sync_copy(hist.at[pl.ds(base, npt)], deg_hbm.at[cid, pl.ds(base, npt)])


def _prep_body(h_ref, Wn_ref, bn_ref, We_ref, be_ref, deg_ref,
               psc_ref, pso_ref, da_ref, db_ref):
  hh = h_ref[...]
  d = hh.shape[1]
  att = jax.nn.softmax(
      jnp.dot(hh, Wn_ref[...], preferred_element_type=jnp.float32)
      + bn_ref[...][None, :], axis=-1)
  u = jnp.dot(hh, We_ref[...][:d], preferred_element_type=jnp.float32)
  v = jnp.dot(hh, We_ref[...][d:], preferred_element_type=jnp.float32)
  onorm = lax.rsqrt(jnp.maximum(deg_ref[0], 1.0))
  psc_ref[...] = att[:, 0] * onorm
  pso_ref[...] = att[:, 1] * onorm
  da_ref[...] = u[:, 1] - u[:, 0]
  db_ref[...] = v[:, 1] - v[:, 0] + (be_ref[1] - be_ref[0])


def _main_body(num_chunks, npt, d,
               hp_hbm, idxp_hbm, psc_hbm, pso_hbm, da_hbm, db_hbm, z2_hbm,
               agg_hbm,
               acc, ii0, ii1, rows0, rows1,
               pva0, pvb0, da0, db0, pva1, pvb1, da1, db1, wbuf,
               sem_g, sem_s0, sem_s1):
  cid = lax.axis_index("c")
  sid = lax.axis_index("s")
  base = sid * npt
  npairs = num_chunks // 2

  pltpu.sync_copy(z2_hbm, acc.at[pl.ds(base, npt)])
  plsc.subcore_barrier()
  isc0 = jnp.broadcast_to(cid == 0, (_LANES,))

  def fire_gathers(ii, rows, pva, pvb, dav, dbv):
    pltpu.async_copy(hp_hbm.at[ii.at[0]], rows, sem_g)
    pltpu.async_copy(psc_hbm.at[ii.at[0]], pva, sem_g)
    pltpu.async_copy(pso_hbm.at[ii.at[0]], pvb, sem_g)
    pltpu.async_copy(da_hbm.at[ii.at[0]], dav, sem_g)
    pltpu.async_copy(db_hbm.at[ii.at[1]], dbv, sem_g)

  def wait_gathers(ii, rows, pva, pvb, dav, dbv):
    pltpu.make_async_copy(hp_hbm.at[ii.at[0]], rows, sem_g).wait()
    pltpu.make_async_copy(psc_hbm.at[ii.at[0]], pva, sem_g).wait()
    pltpu.make_async_copy(pso_hbm.at[ii.at[0]], pvb, sem_g).wait()
    pltpu.make_async_copy(da_hbm.at[ii.at[0]], dav, sem_g).wait()
    pltpu.make_async_copy(db_hbm.at[ii.at[1]], dbv, sem_g).wait()

  def wait_scatter(ii, rows, sem):
    pltpu.make_async_copy(rows, acc.at[ii.at[1]], sem).wait()

  def weigh_and_scale(rows, pva, pvb, dav, dbv):
    for g in range(_CHUNK // _LANES):
      sl = pl.ds(g * _LANES, _LANES)
      z = jnp.exp(dav[sl] + dbv[sl])
      inv = 1.0 / (1.0 + z)
      wbuf[sl] = jnp.where(isc0, pva[sl] * inv, pvb[sl] * (1.0 - inv))

    def row_body(r, rcarry):
      sb = plsc.load_gather(wbuf, [jnp.broadcast_to(r, (_LANES,))])
      for j in range(d // _LANES):
        slj = pl.ds(j * _LANES, _LANES)
        rows[r, slj] = rows[r, slj] * sb
      return rcarry

    lax.fori_loop(0, _CHUNK, row_body, 0)

  pltpu.sync_copy(idxp_hbm.at[sid, 0], ii0)
  fire_gathers(ii0, rows0, pva0, pvb0, da0, db0)

  def pair(i, carry):
    wait_gathers(ii0, rows0, pva0, pvb0, da0, db0)

    @pl.when(i > 0)
    def _():
      wait_scatter(ii1, rows1, sem_s1)

    pltpu.sync_copy(idxp_hbm.at[sid, 2 * i + 1], ii1)
    fire_gathers(ii1, rows1, pva1, pvb1, da1, db1)
    weigh_and_scale(rows0, pva0, pvb0, da0, db0)
    pltpu.async_copy(rows0, acc.at[ii0.at[1]], sem_s0, add=True)

    wait_gathers(ii1, rows1, pva1, pvb1, da1, db1)
    wait_scatter(ii0, rows0, sem_s0)

    @pl.when(i < npairs - 1)
    def _():
      pltpu.sync_copy(idxp_hbm.at[sid, 2 * i + 2], ii0)
      fire_gathers(ii0, rows0, pva0, pvb0, da0, db0)

    weigh_and_scale(rows1, pva1, pvb1, da1, db1)
    pltpu.async_copy(rows1, acc.at[ii1.at[1]], sem_s1, add=True)
    return carry

  lax.fori_loop(0, npairs, pair, 0)
  wait_scatter(ii1, rows1, sem_s1)
  plsc.subcore_barrier()
  pltpu.sync_copy(acc.at[pl.ds(base, npt)], agg_hbm.at[cid, pl.ds(base, npt)])


def _post_body(aggc_ref, aggo_ref, degin_ref, gids_ref,
               Wc_ref, bc_ref, gcg_ref, gcb_ref,
               Wo_ref, bo_ref, gog_ref, gob_ref,
               Wctx_ref, bctx_ref, Wobj_ref, bobj_ref, Wco_ref, bco_ref,
               c_ref, o_ref, co_ref):
  n = aggc_ref.shape[0]
  inorm = lax.rsqrt(jnp.maximum(degin_ref[...], 1.0))

  def gcn_tail(agg, W, b, gamma, beta):
    rst = jnp.dot(agg * inorm[:, None], W,
                  preferred_element_type=jnp.float32) + b[None, :]
    mu = jnp.mean(rst, axis=0, keepdims=True)
    var = jnp.mean((rst - mu) ** 2, axis=0, keepdims=True)
    hbn = (rst - mu) * lax.rsqrt(var + 1e-5) * gamma[None, :] + beta[None, :]
    return jnp.maximum(hbn, 0.0)

  xc = gcn_tail(aggc_ref[...], Wc_ref[...], bc_ref[...], gcg_ref[...],
                gcb_ref[...])
  xo = gcn_tail(aggo_ref[...], Wo_ref[...], bo_ref[...], gog_ref[...],
                gob_ref[...])

  gids = gids_ref[...]
  onehot = (gids[None, :] ==
            lax.broadcasted_iota(jnp.int32, (_NUM_GRAPHS, n), 0)
            ).astype(jnp.float32)
  cnt = jnp.maximum(jnp.sum(onehot, axis=1), 1.0)
  hgc = jnp.dot(onehot, xc, preferred_element_type=jnp.float32) / cnt[:, None]
  hgo = jnp.dot(onehot, xo, preferred_element_type=jnp.float32) / cnt[:, None]
  avg_ctx = jnp.mean(hgc, axis=0, keepdims=True)
  c_ref[...] = jnp.dot(hgc, Wctx_ref[...],
                       preferred_element_type=jnp.float32) + bctx_ref[...][None, :]
  o_ref[...] = jnp.dot(hgo, Wobj_ref[...],
                       preferred_element_type=jnp.float32) + bobj_ref[...][None, :]
  co_ref[...] = jnp.dot(avg_ctx + hgo, Wco_ref[...],
                        preferred_element_type=jnp.float32) + bco_ref[...][None, :]


def kernel(h, Wn, bn, We, be, Wc_gcn, bc_gcn, gc_gamma, gc_beta,
           Wo_gcn, bo_gcn, go_gamma, go_beta, Wctx, bctx, Wobj, bobj,
           Wco, bco, edge_index, graph_ids):
  n, d = h.shape
  e = edge_index.shape[1]
  f32 = jnp.float32

  np_pad = ((n + _TILES * _LANES - 1) // (_TILES * _LANES)) * (_TILES * _LANES)
  gsz = _TILES * _CHUNK * _GROUP
  ngroups = (e + gsz - 1) // gsz
  num_chunks = ngroups * _GROUP
  pe = _TILES * num_chunks * _CHUNK
  npt = np_pad // _TILES

  hp = jnp.pad(h, ((0, np_pad - n), (0, 0)))
  pad_idx = n + (jnp.arange(pe - e, dtype=jnp.int32) % (np_pad - n))
  rowr = jnp.concatenate([edge_index[0], pad_idx]).reshape(
      _TILES, num_chunks, _CHUNK)
  colr = jnp.concatenate([edge_index[1], pad_idx]).reshape(
      _TILES, num_chunks, _CHUNK)
  idx2 = jnp.stack([rowr, colr])
  idxm = jnp.stack([rowr, colr], axis=2)

  z1 = jnp.zeros((npt,), f32)
  z2 = jnp.zeros((npt, d), f32)

  mesh = plsc.VectorSubcoreMesh(core_axis_name="c", subcore_axis_name="s")

  deg = pl.kernel(
      functools.partial(_deg_body, num_chunks, npt),
      out_type=jax.ShapeDtypeStruct((2, np_pad), f32),
      mesh=mesh,
      scratch_types=[
          pltpu.VMEM_SHARED((np_pad,), f32),
          pltpu.VMEM((num_chunks, _CHUNK), jnp.int32),
          pltpu.VMEM((_CHUNK,), f32),
          pltpu.SemaphoreType.DMA,
      ],
  )(idx2, z1)

  tab_shape = jax.ShapeDtypeStruct((np_pad,), f32)
  psc, pso, da, db = pl.pallas_call(
      _prep_body,
      out_shape=(tab_shape, tab_shape, tab_shape, tab_shape),
  )(hp, Wn, bn, We, be, deg)

  agg = pl.kernel(
      functools.partial(_main_body, num_chunks, npt, d),
      out_type=jax.ShapeDtypeStruct((2, np_pad, d), f32),
      mesh=mesh,
      compiler_params=pltpu.CompilerParams(needs_layout_passes=False),
      scratch_types=[
          pltpu.VMEM_SHARED((np_pad, d), f32),
          pltpu.VMEM((2, _CHUNK), jnp.int32),
          pltpu.VMEM((2, _CHUNK), jnp.int32),
          pltpu.VMEM((_CHUNK, d), f32),
          pltpu.VMEM((_CHUNK, d), f32),
          pltpu.VMEM((_CHUNK,), f32),
          pltpu.VMEM((_CHUNK,), f32),
          pltpu.VMEM((_CHUNK,), f32),
          pltpu.VMEM((_CHUNK,), f32),
          pltpu.VMEM((_CHUNK,), f32),
          pltpu.VMEM((_CHUNK,), f32),
          pltpu.VMEM((_CHUNK,), f32),
          pltpu.VMEM((_CHUNK,), f32),
          pltpu.VMEM((_CHUNK,), f32),
          pltpu.SemaphoreType.DMA,
          pltpu.SemaphoreType.DMA,
          pltpu.SemaphoreType.DMA,
      ],
  )(hp, idxm, psc, pso, da, db, z2)

  out_shape = jax.ShapeDtypeStruct((_NUM_GRAPHS, Wctx.shape[1]), f32)
  c_out, o_out, co_out = pl.pallas_call(
      _post_body,
      out_shape=(out_shape, out_shape, out_shape),
  )(agg[0, :n], agg[1, :n], deg[1, :n], graph_ids,
    Wc_gcn, bc_gcn, gc_gamma, gc_beta,
    Wo_gcn, bo_gcn, go_gamma, go_beta,
    Wctx, bctx, Wobj, bobj, Wco, bco)
  return (c_out, o_out, co_out)
